# baseline (speedup 1.0000x reference)
import functools

import jax
import jax.numpy as jnp
from jax import lax
from jax.experimental import pallas as pl
from jax.experimental.pallas import tpu as pltpu
from jax.experimental.pallas import tpu_sc as plsc

_BLOCK = 512
_SPLIT_BLOCKS = 10
_CHUNK = 8


def _body_left(w_ref, nf_ref, nb_ref, out_ref):
    nb = nb_ref[...]
    mean = jnp.sum(nb, axis=1) * (1.0 / nb.shape[1])
    x = jnp.concatenate([nf_ref[...], mean], axis=1)
    y = jax.lax.dot_general(
        w_ref[...], x, (((1,), (1,)), ((), ())),
        preferred_element_type=jnp.float32)
    out_ref[...] = jnp.maximum(y, 0.0)


def _body_right(w_ref, nf_ref, sum_ref, out_ref):
    x = jnp.concatenate([nf_ref[...], sum_ref[...]], axis=1)
    y = jax.lax.dot_general(
        w_ref[...], x, (((1,), (1,)), ((), ())),
        preferred_element_type=jnp.float32)
    out_ref[...] = jnp.maximum(y, 0.0)


def _sc_sums(neighbor_feats, n_off, n_nodes, s, f):
    info = plsc.get_sparse_core_info()
    nc, ns = info.num_cores, info.num_subcores
    nw = nc * ns
    c = _CHUNK
    nch_total = n_nodes // c
    r = c * s
    g = f // 16
    mesh = plsc.VectorSubcoreMesh(core_axis_name="c", subcore_axis_name="s")

    @functools.partial(
        pl.kernel, mesh=mesh,
        out_type=jax.ShapeDtypeStruct((n_nodes, f), jnp.float32),
        scratch_types=[
            pltpu.VMEM((r, f), jnp.float32),
            pltpu.VMEM((r, f), jnp.float32),
            pltpu.VMEM((c, f), jnp.float32),
            pltpu.SemaphoreType.DMA,
            pltpu.SemaphoreType.DMA,
        ],
    )
    def k(neigh_hbm, out_hbm, buf0, buf1, outb, sem0, sem1):
        wid = lax.axis_index("s") * nc + lax.axis_index("c")
        nch = (nch_total - wid + nw - 1) // nw

        def rowbase(j):
            return (n_off + (wid + j * nw) * c) * s

        def start(j, buf, sem):
            pltpu.async_copy(neigh_hbm.at[pl.ds(rowbase(j), r)], buf, sem)

        def wait(j, buf, sem):
            pltpu.make_async_copy(
                neigh_hbm.at[pl.ds(rowbase(j), r)], buf, sem).wait()

        def compute(j, buf):
            def node_body(i, carry):
                for gg in range(g):
                    a = buf[i * s, pl.ds(gg * 16, 16)]
                    for rr in range(1, s):
                        a = a + buf[i * s + rr, pl.ds(gg * 16, 16)]
                    outb[i, pl.ds(gg * 16, 16)] = a
                return carry
            lax.fori_loop(0, c, node_body, 0)
            pltpu.sync_copy(outb, out_hbm.at[pl.ds((wid + j * nw) * c, c)])

        start(0, buf0, sem0)

        def body(j2, carry):
            j = j2 * 2
            wait(j, buf0, sem0)

            @pl.when(j + 1 < nch)
            def _():
                start(j + 1, buf1, sem1)

            compute(j, buf0)

            @pl.when(j + 1 < nch)
            def _():
                wait(j + 1, buf1, sem1)

                @pl.when(j + 2 < nch)
                def _():
                    start(j + 2, buf0, sem0)

                compute(j + 1, buf1)

            return carry

        lax.fori_loop(0, (nch + 1) // 2, body, 0)

    return k(neighbor_feats)


def kernel(node_feats, neighbor_feats, weight_1, weight_2, node_count):
    n, f = node_feats.shape
    s = neighbor_feats.shape[0] // n
    e = weight_1.shape[0]
    b = _BLOCK
    nt = min(_SPLIT_BLOCKS * b, n)
    nright = n - nt

    scale = jnp.float32(node_count) / jnp.float32(n)
    z = jnp.zeros((e, f), jnp.float32)
    w_left = jnp.concatenate(
        [jnp.concatenate([weight_1, z], axis=1),
         jnp.concatenate([z, weight_2], axis=1)], axis=0) * scale
    w_right = jnp.concatenate(
        [jnp.concatenate([weight_1, z], axis=1),
         jnp.concatenate([z, weight_2 * (1.0 / s)], axis=1)], axis=0) * scale
    nb3 = neighbor_feats.reshape(n, s, f)

    sums = _sc_sums(neighbor_feats, nt, nright, s, f)

    out_left = pl.pallas_call(
        _body_left,
        grid=(nt // b,),
        in_specs=[
            pl.BlockSpec((2 * e, 2 * f), lambda i: (0, 0)),
            pl.BlockSpec((b, f), lambda i: (i, 0)),
            pl.BlockSpec((b, s, f), lambda i: (i, 0, 0)),
        ],
        out_specs=pl.BlockSpec((2 * e, b), lambda i: (0, i)),
        out_shape=jax.ShapeDtypeStruct((2 * e, nt), jnp.float32),
    )(w_left, node_feats, nb3)

    nlb = nt // b
    out_right = pl.pallas_call(
        _body_right,
        grid=((nright + b - 1) // b,),
        in_specs=[
            pl.BlockSpec((2 * e, 2 * f), lambda i: (0, 0)),
            pl.BlockSpec((b, f), lambda i: (i + nlb, 0)),
            pl.BlockSpec((b, f), lambda i: (i, 0)),
        ],
        out_specs=pl.BlockSpec((2 * e, b), lambda i: (0, i)),
        out_shape=jax.ShapeDtypeStruct((2 * e, nright), jnp.float32),
    )(w_right, node_feats, sums)

    return jnp.concatenate([out_left, out_right], axis=1)


# speedup vs baseline: 1.1206x; 1.1206x over previous
import functools

import jax
import jax.numpy as jnp
from jax import lax
from jax.experimental import pallas as pl
from jax.experimental.pallas import tpu as pltpu
from jax.experimental.pallas import tpu_sc as plsc

_BLOCK = 512
_SPLIT_BLOCKS = 14
_CHUNK = 8


def _body_left(w_ref, nf_ref, nb_ref, out_ref):
    nb = nb_ref[...]
    mean = jnp.sum(nb, axis=1) * (1.0 / nb.shape[1])
    x = jnp.concatenate([nf_ref[...], mean], axis=1)
    y = jax.lax.dot_general(
        w_ref[...], x, (((1,), (1,)), ((), ())),
        preferred_element_type=jnp.float32)
    out_ref[...] = jnp.maximum(y, 0.0)


def _body_right(w_ref, nf_ref, sum_ref, out_ref):
    x = jnp.concatenate([nf_ref[...], sum_ref[...]], axis=1)
    y = jax.lax.dot_general(
        w_ref[...], x, (((1,), (1,)), ((), ())),
        preferred_element_type=jnp.float32)
    out_ref[...] = jnp.maximum(y, 0.0)


def _sc_sums(neighbor_feats, n_off, n_nodes, s, f):
    info = plsc.get_sparse_core_info()
    nc, ns = info.num_cores, info.num_subcores
    nw = nc * ns
    c = _CHUNK
    nch_total = n_nodes // c
    r = c * s
    g = f // 16
    mesh = plsc.VectorSubcoreMesh(core_axis_name="c", subcore_axis_name="s")

    @functools.partial(
        pl.kernel, mesh=mesh,
        out_type=jax.ShapeDtypeStruct((n_nodes, f), jnp.float32),
        scratch_types=[
            pltpu.VMEM((r, f), jnp.float32),
            pltpu.VMEM((r, f), jnp.float32),
            pltpu.VMEM((c, f), jnp.float32),
            pltpu.SemaphoreType.DMA,
            pltpu.SemaphoreType.DMA,
        ],
    )
    def k(neigh_hbm, out_hbm, buf0, buf1, outb, sem0, sem1):
        wid = lax.axis_index("s") * nc + lax.axis_index("c")
        nch = (nch_total - wid + nw - 1) // nw

        def rowbase(j):
            return (n_off + (wid + j * nw) * c) * s

        def start(j, buf, sem):
            pltpu.async_copy(neigh_hbm.at[pl.ds(rowbase(j), r)], buf, sem)

        def wait(j, buf, sem):
            pltpu.make_async_copy(
                neigh_hbm.at[pl.ds(rowbase(j), r)], buf, sem).wait()

        def compute(j, buf):
            def node_body(i, carry):
                for gg in range(g):
                    a = buf[i * s, pl.ds(gg * 16, 16)]
                    for rr in range(1, s):
                        a = a + buf[i * s + rr, pl.ds(gg * 16, 16)]
                    outb[i, pl.ds(gg * 16, 16)] = a
                return carry
            lax.fori_loop(0, c, node_body, 0)
            pltpu.sync_copy(outb, out_hbm.at[pl.ds((wid + j * nw) * c, c)])

        start(0, buf0, sem0)

        def body(j2, carry):
            j = j2 * 2
            wait(j, buf0, sem0)

            @pl.when(j + 1 < nch)
            def _():
                start(j + 1, buf1, sem1)

            compute(j, buf0)

            @pl.when(j + 1 < nch)
            def _():
                wait(j + 1, buf1, sem1)

                @pl.when(j + 2 < nch)
                def _():
                    start(j + 2, buf0, sem0)

                compute(j + 1, buf1)

            return carry

        lax.fori_loop(0, (nch + 1) // 2, body, 0)

    return k(neighbor_feats)


def kernel(node_feats, neighbor_feats, weight_1, weight_2, node_count):
    n, f = node_feats.shape
    s = neighbor_feats.shape[0] // n
    e = weight_1.shape[0]
    b = _BLOCK
    nt = min(_SPLIT_BLOCKS * b, n)
    nright = n - nt

    scale = jnp.float32(node_count) / jnp.float32(n)
    z = jnp.zeros((e, f), jnp.float32)
    w_left = jnp.concatenate(
        [jnp.concatenate([weight_1, z], axis=1),
         jnp.concatenate([z, weight_2], axis=1)], axis=0) * scale
    w_right = jnp.concatenate(
        [jnp.concatenate([weight_1, z], axis=1),
         jnp.concatenate([z, weight_2 * (1.0 / s)], axis=1)], axis=0) * scale
    nb3 = neighbor_feats.reshape(n, s, f)

    sums = _sc_sums(neighbor_feats, nt, nright, s, f)

    out_left = pl.pallas_call(
        _body_left,
        grid=(nt // b,),
        in_specs=[
            pl.BlockSpec((2 * e, 2 * f), lambda i: (0, 0)),
            pl.BlockSpec((b, f), lambda i: (i, 0)),
            pl.BlockSpec((b, s, f), lambda i: (i, 0, 0)),
        ],
        out_specs=pl.BlockSpec((2 * e, b), lambda i: (0, i)),
        out_shape=jax.ShapeDtypeStruct((2 * e, nt), jnp.float32),
    )(w_left, node_feats, nb3)

    nlb = nt // b
    out_right = pl.pallas_call(
        _body_right,
        grid=((nright + b - 1) // b,),
        in_specs=[
            pl.BlockSpec((2 * e, 2 * f), lambda i: (0, 0)),
            pl.BlockSpec((b, f), lambda i: (i + nlb, 0)),
            pl.BlockSpec((b, f), lambda i: (i, 0)),
        ],
        out_specs=pl.BlockSpec((2 * e, b), lambda i: (0, i)),
        out_shape=jax.ShapeDtypeStruct((2 * e, nright), jnp.float32),
    )(w_right, node_feats, sums)

    return jnp.concatenate([out_left, out_right], axis=1)


# speedup vs baseline: 1.2546x; 1.1195x over previous
import functools

import jax
import jax.numpy as jnp
from jax import lax
from jax.experimental import pallas as pl
from jax.experimental.pallas import tpu as pltpu
from jax.experimental.pallas import tpu_sc as plsc

_BLOCK = 512
_SPLIT_BLOCKS = 14
_CHUNK = 8


def _body_left(w_ref, nf_ref, nb_ref, out_ref):
    nb = nb_ref[...]
    mean = jnp.sum(nb, axis=1) * (1.0 / nb.shape[1])
    x = jnp.concatenate([nf_ref[...], mean], axis=1)
    y = jax.lax.dot_general(
        w_ref[...], x, (((1,), (1,)), ((), ())),
        preferred_element_type=jnp.float32)
    out_ref[...] = jnp.maximum(y, 0.0)


def _body_right(w_ref, nf_ref, sum_ref, left_ref, out_ref):
    del left_ref
    x = jnp.concatenate([nf_ref[...], sum_ref[...]], axis=1)
    y = jax.lax.dot_general(
        w_ref[...], x, (((1,), (1,)), ((), ())),
        preferred_element_type=jnp.float32)
    out_ref[...] = jnp.maximum(y, 0.0)


def _sc_sums(neighbor_feats, n_off, n_nodes, s, f):
    info = plsc.get_sparse_core_info()
    nc, ns = info.num_cores, info.num_subcores
    nw = nc * ns
    c = _CHUNK
    nch_total = n_nodes // c
    r = c * s
    g = f // 16
    mesh = plsc.VectorSubcoreMesh(core_axis_name="c", subcore_axis_name="s")

    @functools.partial(
        pl.kernel, mesh=mesh,
        out_type=jax.ShapeDtypeStruct((n_nodes, f), jnp.float32),
        scratch_types=[
            pltpu.VMEM((r, f), jnp.float32),
            pltpu.VMEM((r, f), jnp.float32),
            pltpu.VMEM((c, f), jnp.float32),
            pltpu.SemaphoreType.DMA,
            pltpu.SemaphoreType.DMA,
        ],
    )
    def k(neigh_hbm, out_hbm, buf0, buf1, outb, sem0, sem1):
        wid = lax.axis_index("s") * nc + lax.axis_index("c")
        nch = (nch_total - wid + nw - 1) // nw

        def rowbase(j):
            return (n_off + (wid + j * nw) * c) * s

        def start(j, buf, sem):
            pltpu.async_copy(neigh_hbm.at[pl.ds(rowbase(j), r)], buf, sem)

        def wait(j, buf, sem):
            pltpu.make_async_copy(
                neigh_hbm.at[pl.ds(rowbase(j), r)], buf, sem).wait()

        def compute(j, buf):
            def node_body(i, carry):
                for gg in range(g):
                    a = buf[i * s, pl.ds(gg * 16, 16)]
                    for rr in range(1, s):
                        a = a + buf[i * s + rr, pl.ds(gg * 16, 16)]
                    outb[i, pl.ds(gg * 16, 16)] = a
                return carry
            lax.fori_loop(0, c, node_body, 0)
            pltpu.sync_copy(outb, out_hbm.at[pl.ds((wid + j * nw) * c, c)])

        start(0, buf0, sem0)

        def body(j2, carry):
            j = j2 * 2
            wait(j, buf0, sem0)

            @pl.when(j + 1 < nch)
            def _():
                start(j + 1, buf1, sem1)

            compute(j, buf0)

            @pl.when(j + 1 < nch)
            def _():
                wait(j + 1, buf1, sem1)

                @pl.when(j + 2 < nch)
                def _():
                    start(j + 2, buf0, sem0)

                compute(j + 1, buf1)

            return carry

        lax.fori_loop(0, (nch + 1) // 2, body, 0)

    return k(neighbor_feats)


def kernel(node_feats, neighbor_feats, weight_1, weight_2, node_count):
    n, f = node_feats.shape
    s = neighbor_feats.shape[0] // n
    e = weight_1.shape[0]
    b = _BLOCK
    nt = min(_SPLIT_BLOCKS * b, n)
    nright = n - nt

    scale = jnp.float32(node_count) / jnp.float32(n)
    z = jnp.zeros((e, f), jnp.float32)
    w_left = jnp.concatenate(
        [jnp.concatenate([weight_1, z], axis=1),
         jnp.concatenate([z, weight_2], axis=1)], axis=0) * scale
    w_right = jnp.concatenate(
        [jnp.concatenate([weight_1, z], axis=1),
         jnp.concatenate([z, weight_2 * (1.0 / s)], axis=1)], axis=0) * scale
    nb3 = neighbor_feats.reshape(n, s, f)

    sums = _sc_sums(neighbor_feats, nt, nright, s, f)

    out_left = pl.pallas_call(
        _body_left,
        grid=(nt // b,),
        in_specs=[
            pl.BlockSpec((2 * e, 2 * f), lambda i: (0, 0)),
            pl.BlockSpec((b, f), lambda i: (i, 0)),
            pl.BlockSpec((b, s, f), lambda i: (i, 0, 0)),
        ],
        out_specs=pl.BlockSpec((2 * e, b), lambda i: (0, i)),
        out_shape=jax.ShapeDtypeStruct((2 * e, n), jnp.float32),
    )(w_left, node_feats, nb3)

    nlb = nt // b
    out = pl.pallas_call(
        _body_right,
        grid=((nright + b - 1) // b,),
        in_specs=[
            pl.BlockSpec((2 * e, 2 * f), lambda i: (0, 0)),
            pl.BlockSpec((b, f), lambda i: (i + nlb, 0)),
            pl.BlockSpec((b, f), lambda i: (i, 0)),
            pl.BlockSpec(memory_space=pl.ANY),
        ],
        out_specs=pl.BlockSpec((2 * e, b), lambda i: (0, i + nlb)),
        out_shape=jax.ShapeDtypeStruct((2 * e, n), jnp.float32),
        input_output_aliases={3: 0},
    )(w_right, node_feats, sums, out_left)

    return out


# speedup vs baseline: 1.7294x; 1.3785x over previous
import jax
import jax.numpy as jnp
from jax.experimental import pallas as pl

_BLOCK = 512


def _body(w_ref, nf_ref, nb_ref, out_ref):
    nb = nb_ref[...]
    mean = jnp.sum(nb, axis=1) * (1.0 / nb.shape[1])
    x = jnp.concatenate([nf_ref[...], mean], axis=1)
    y = jax.lax.dot_general(
        w_ref[...], x, (((1,), (1,)), ((), ())),
        preferred_element_type=jnp.float32)
    out_ref[...] = jnp.maximum(y, 0.0)


def kernel(node_feats, neighbor_feats, weight_1, weight_2, node_count):
    n, f = node_feats.shape
    s = neighbor_feats.shape[0] // n
    e = weight_1.shape[0]
    scale = jnp.float32(node_count) / jnp.float32(n)
    z = jnp.zeros((e, f), jnp.float32)
    w = jnp.concatenate(
        [jnp.concatenate([weight_1, z], axis=1),
         jnp.concatenate([z, weight_2], axis=1)], axis=0) * scale
    nb3 = neighbor_feats.reshape(n, s, f)
    b = _BLOCK
    grid = (n + b - 1) // b
    return pl.pallas_call(
        _body,
        grid=(grid,),
        in_specs=[
            pl.BlockSpec((2 * e, 2 * f), lambda i: (0, 0)),
            pl.BlockSpec((b, f), lambda i: (i, 0)),
            pl.BlockSpec((b, s, f), lambda i: (i, 0, 0)),
        ],
        out_specs=pl.BlockSpec((2 * e, b), lambda i: (0, i)),
        out_shape=jax.ShapeDtypeStruct((2 * e, n), jnp.float32),
    )(w, node_feats, nb3)
